# Initial kernel scaffold; baseline (speedup 1.0000x reference)
import jax
import jax.numpy as jnp
from jax.experimental import pallas as pl


def kernel(x, edge_index, edge_weight, W):
    raise NotImplementedError("write your pallas kernel here")


# speedup vs baseline: 5.2229x; 5.2229x over previous
import functools

import jax
import jax.numpy as jnp
from jax import lax
from jax.experimental import pallas as pl
from jax.experimental.pallas import tpu as pltpu
from jax.experimental.pallas import tpu_sc as plsc

NC = 2
NS = 16
LANES = 16


@functools.lru_cache(maxsize=None)
def _build_sc_scatter(N, D, E, CH):
    NW = NC * NS
    E_W = E // NW
    NCHUNK = E_W // CH
    ROWS_W = N // NS
    ZR = 125
    mesh = plsc.VectorSubcoreMesh(
        core_axis_name="c", subcore_axis_name="s",
        num_cores=NC, num_subcores=NS)

    @functools.partial(
        pl.kernel,
        out_type=jax.ShapeDtypeStruct((NC, N, D), jnp.float32),
        mesh=mesh,
        scratch_types=[
            pltpu.VMEM_SHARED((N, D), jnp.float32),
            pltpu.VMEM((CH, D), jnp.float32),
            pltpu.VMEM((CH,), jnp.int32),
            pltpu.VMEM((CH,), jnp.int32),
            pltpu.VMEM((CH,), jnp.float32),
            pltpu.SemaphoreType.DMA,
        ],
        compiler_params=pltpu.CompilerParams(use_tc_tiling_on_sc=False,
                                              needs_layout_passes=False),
    )
    def sc_scatter(src_hbm, dst_hbm, w_hbm, x_hbm, out_hbm,
                   acc, rows, sidx, didx, wbuf, sem):
        c = lax.axis_index("c")
        s = lax.axis_index("s")
        wid = s * NC + c

        def zero_body(r, carry):
            for f in range(D // LANES):
                rows[r, pl.ds(f * LANES, LANES)] = jnp.zeros((LANES,), jnp.float32)
            return carry
        lax.fori_loop(0, ZR, zero_body, 0)
        slab = s * ROWS_W
        for k in range(ROWS_W // ZR):
            pltpu.sync_copy(rows.at[pl.ds(0, ZR)],
                            acc.at[pl.ds(slab + k * ZR, ZR)])
        plsc.subcore_barrier()

        def chunk_body(k, carry):
            base = wid * E_W + k * CH
            pltpu.sync_copy(src_hbm.at[pl.ds(base, CH)], sidx)
            pltpu.sync_copy(dst_hbm.at[pl.ds(base, CH)], didx)
            pltpu.sync_copy(w_hbm.at[pl.ds(base, CH)], wbuf)
            pltpu.async_copy(x_hbm.at[sidx], rows, sem).wait()

            def edge_body(e, ecarry):
                w16 = plsc.load_gather(wbuf, [jnp.full((LANES,), e, jnp.int32)])
                for f in range(D // LANES):
                    sl = pl.ds(f * LANES, LANES)
                    rows[e, sl] = rows[e, sl] * w16
                return ecarry
            lax.fori_loop(0, CH, edge_body, 0)

            pltpu.sync_copy(rows, acc.at[didx], add=True)
            return carry
        lax.fori_loop(0, NCHUNK, chunk_body, 0)
        plsc.subcore_barrier()

        pltpu.sync_copy(acc.at[pl.ds(slab, ROWS_W)],
                        out_hbm.at[c, pl.ds(slab, ROWS_W)])

    return sc_scatter


@functools.lru_cache(maxsize=None)
def _build_tc_matmul(N, D, DO, BLK):
    def tc_body(p_ref, w_ref, o_ref):
        p = p_ref[0] + p_ref[1]
        o_ref[...] = jnp.maximum(
            jnp.dot(p, w_ref[...], preferred_element_type=jnp.float32), 0.0)

    return pl.pallas_call(
        tc_body,
        grid=(N // BLK,),
        in_specs=[
            pl.BlockSpec((2, BLK, D), lambda i: (0, i, 0)),
            pl.BlockSpec((D, DO), lambda i: (0, 0)),
        ],
        out_specs=pl.BlockSpec((BLK, DO), lambda i: (i, 0)),
        out_shape=jax.ShapeDtypeStruct((N, DO), jnp.float32),
    )


def kernel(x, edge_index, edge_weight, W):
    N, D = x.shape
    DO = W.shape[1]
    E = edge_weight.shape[0]
    src = edge_index[1]
    dst = edge_index[0]
    partial = _build_sc_scatter(N, D, E, 200)(src, dst, edge_weight, x)
    return _build_tc_matmul(N, D, DO, 1000)(partial, W)
